# baseline (speedup 1.0000x reference)
import functools

import jax
import jax.numpy as jnp
import numpy as np
from jax import lax
from jax.experimental import pallas as pl
from jax.experimental.pallas import tpu as pltpu
from jax.experimental.pallas import tpu_sc as plsc

N_KEEP = 48
ROWS = 16
H = 224
W = 224
C = 96
L = H * W * C
LANES = 16
VPW = C // LANES
PLANES_PER_PIECE = 2
NPIECES = (H // 2) // PLANES_PER_PIECE
NEG_INF = float("-inf")


def _i32(v):
    v &= 0xFFFFFFFF
    return np.int32(v - (1 << 32) if v >= (1 << 31) else v)


MIN32 = _i32(0x80000000)
M31 = _i32(0x7FFFFFFF)


def _bmerge(a, b):
    rb = jnp.flip(b)
    lo = jnp.minimum(a, rb)
    hi = jnp.maximum(a, rb)
    return jnp.sort(lo), jnp.sort(hi)


def _make_insert(th_v):
    def _insert(v, c):
        t0, t1, t2, _ = c
        s = jnp.sort(v)
        _, hi = _bmerge(s, t0)
        t0, hi = _bmerge(hi, t1)
        t1, t2 = _bmerge(hi, t2)
        theta = jnp.full((LANES,), t0[0], jnp.float32)
        return (t0, t1, t2, theta)
    return _insert


def _any_gt(v, theta):
    return plsc.all_reduce_population_count(v > theta)[0] > 0


def _process_piece(buf, th_v, carry):
    insert = _make_insert(th_v)

    GR = 8
    for p in range(PLANES_PER_PIECE):
        def grp_body(wg, c, p=p):
            w = GR * wg
            rm = []
            for dw in range(GR):
                m = buf[p, w + dw, pl.ds(0, LANES)]
                for u in range(1, VPW):
                    m = jnp.maximum(m, buf[p, w + dw, pl.ds(u * LANES, LANES)])
                rm.append(m)
            gm = rm[0]
            for dw in range(1, GR):
                gm = jnp.maximum(gm, rm[dw])

            def hit(cc):
                for dw in range(GR):
                    def row_hit(c3, dw=dw):
                        for u in range(VPW):
                            v = buf[p, w + dw, pl.ds(u * LANES, LANES)]
                            c3 = lax.cond(_any_gt(v, c3[3]),
                                          functools.partial(insert, v),
                                          lambda c4: c4, c3)
                        return c3
                    cc = lax.cond(_any_gt(rm[dw], cc[3]), row_hit,
                                  lambda c3: c3, cc)
                return cc

            return lax.cond(_any_gt(gm, c[3]), hit, lambda c2: c2, c)

        carry = lax.fori_loop(0, W // GR, grp_body, carry)
    return carry


GROUPS = 1
GR_ROWS = ROWS // GROUPS
SHARDS = 32 // GR_ROWS
SH_PLANES = H // SHARDS
NPIECES = SH_PLANES // PLANES_PER_PIECE


def _sc_body(x_hbm, cand_hbm, buf0, buf1, out_v, sem0, sem1, *, g):
    wid = lax.axis_index("s") * 2 + lax.axis_index("c")
    row_local = wid // SHARDS
    shard = wid - row_local * SHARDS
    row = g * GR_ROWS + row_local
    base0 = shard * SH_PLANES

    def src(i):
        return x_hbm.at[row, pl.ds(base0 + i * PLANES_PER_PIECE,
                                   PLANES_PER_PIECE), :, :]

    pltpu.async_copy(src(0), buf0, sem0)
    pltpu.async_copy(src(1), buf1, sem1)

    neg = jnp.full((LANES,), NEG_INF, jnp.float32)
    carry0 = (neg, neg, neg, neg)

    def pair_body(gi, carry):
        i0 = 2 * gi
        pltpu.make_async_copy(src(i0), buf0, sem0).wait()
        carry = _process_piece(buf0, out_v, carry)

        @pl.when(i0 + 2 < NPIECES)
        def _():
            pltpu.async_copy(src(i0 + 2), buf0, sem0)

        pltpu.make_async_copy(src(i0 + 1), buf1, sem1).wait()
        carry = _process_piece(buf1, out_v, carry)

        @pl.when(i0 + 3 < NPIECES)
        def _():
            pltpu.async_copy(src(i0 + 3), buf1, sem1)

        return carry

    t0, t1, t2, _ = lax.fori_loop(0, NPIECES // 2, pair_body, carry0)
    out_v[pl.ds(0, LANES)] = jnp.full((LANES,), NEG_INF, jnp.float32)
    out_v[pl.ds(LANES, LANES)] = t0
    out_v[pl.ds(2 * LANES, LANES)] = t1
    out_v[pl.ds(3 * LANES, LANES)] = t2
    pltpu.sync_copy(out_v, cand_hbm.at[row_local, pl.ds(shard * 64, 64)])


def _sc_candidates(x4d, g):
    mesh = plsc.VectorSubcoreMesh(core_axis_name="c", subcore_axis_name="s")
    fn = pl.kernel(
        functools.partial(_sc_body, g=g),
        out_type=jax.ShapeDtypeStruct((GR_ROWS, SHARDS * 64), jnp.float32),
        mesh=mesh,
        compiler_params=pltpu.CompilerParams(needs_layout_passes=False,
                                             use_tc_tiling_on_sc=True),
        scratch_types=[
            pltpu.VMEM((PLANES_PER_PIECE, W, C), jnp.float32),
            pltpu.VMEM((PLANES_PER_PIECE, W, C), jnp.float32),
            pltpu.VMEM((64,), jnp.float32),
            pltpu.SemaphoreType.DMA,
            pltpu.SemaphoreType.DMA,
        ],
    )
    return fn(x4d)


H_BLK = 56
NSPLIT = 224 // H_BLK


NCVEC = SHARDS * 64 // 128


def _tc_mask_body(x_ref, c_ref, o_ref, tau_ref):
    j = pl.program_id(1)

    @pl.when(j == 0)
    def _():
        c = c_ref[0]
        u = lax.bitcast_convert_type(c, jnp.int32)
        skey = jnp.where(u >= 0, u, u ^ M31)
        t = jnp.int32(0)
        for b in range(31, -1, -1):
            cu = t | _i32(1 << b)
            scand = cu ^ MIN32
            cnt = jnp.sum((skey >= scand).astype(jnp.int32))
            t = jnp.where(cnt >= N_KEEP, cu, t)
        st = t ^ MIN32
        ub = jnp.where(st >= 0, st, st ^ M31)
        tauv = lax.bitcast_convert_type(
            jnp.broadcast_to(ub, (1, 128)), jnp.float32)
        tau_ref[...] = tauv

    tau = tau_ref[0, 0]
    x = x_ref[0]
    o_ref[0] = x * (x >= tau).astype(jnp.float32)


def _tc_mask_body_prev(x_ref, c_ref, prev_ref, o_ref, tau_ref):
    del prev_ref
    _tc_mask_body(x_ref, c_ref, o_ref, tau_ref)


def _tc_mask(x4d, cand3d, g, prev=None):
    in_specs = [
        pl.BlockSpec((1, H_BLK, 224, 96),
                     lambda r, j: (g * GR_ROWS + r, j, 0, 0)),
        pl.BlockSpec((1, NCVEC, 128), lambda r, j: (r, 0, 0)),
    ]
    args = [x4d, cand3d]
    body = _tc_mask_body
    io_alias = {}
    if prev is not None:
        in_specs.append(pl.BlockSpec(memory_space=pl.ANY))
        args.append(prev)
        body = _tc_mask_body_prev
        io_alias = {2: 0}
    return pl.pallas_call(
        body,
        grid=(GR_ROWS, NSPLIT),
        in_specs=in_specs,
        out_specs=pl.BlockSpec((1, H_BLK, 224, 96),
                               lambda r, j: (g * GR_ROWS + r, j, 0, 0)),
        out_shape=jax.ShapeDtypeStruct(x4d.shape, jnp.float32),
        scratch_shapes=[pltpu.VMEM((1, 128), jnp.float32)],
        input_output_aliases=io_alias,
    )(*args)


def kernel(inputs):
    cands = [_sc_candidates(inputs, g) for g in range(GROUPS)]
    full = None
    for g in range(GROUPS):
        c3 = cands[g].reshape(GR_ROWS, NCVEC, 128)
        full = _tc_mask(inputs, c3, g, prev=full)
    return full


# speedup vs baseline: 1.0286x; 1.0286x over previous
import functools

import jax
import jax.numpy as jnp
import numpy as np
from jax import lax
from jax.experimental import pallas as pl
from jax.experimental.pallas import tpu as pltpu
from jax.experimental.pallas import tpu_sc as plsc

N_KEEP = 48
ROWS = 16
H = 224
W = 224
C = 96
L = H * W * C
LANES = 16
VPW = C // LANES
PLANES_PER_PIECE = 2
NPIECES = (H // 2) // PLANES_PER_PIECE
NEG_INF = float("-inf")


def _i32(v):
    v &= 0xFFFFFFFF
    return np.int32(v - (1 << 32) if v >= (1 << 31) else v)


MIN32 = _i32(0x80000000)
M31 = _i32(0x7FFFFFFF)


def _bmerge(a, b):
    rb = jnp.flip(b)
    lo = jnp.minimum(a, rb)
    hi = jnp.maximum(a, rb)
    return jnp.sort(lo), jnp.sort(hi)


def _make_insert(th_v):
    def _insert(v, c):
        t0, t1, t2, _ = c
        s = jnp.sort(v)
        _, hi = _bmerge(s, t0)
        t0, hi = _bmerge(hi, t1)
        t1, t2 = _bmerge(hi, t2)
        theta = jnp.full((LANES,), t0[0], jnp.float32)
        return (t0, t1, t2, theta)
    return _insert


def _any_gt(v, theta):
    return plsc.all_reduce_population_count(v > theta)[0] > 0


def _process_piece(buf, th_v, carry):
    insert = _make_insert(th_v)

    GR = 4
    for p in range(PLANES_PER_PIECE):
        def grp_body(wg, c, p=p):
            w = GR * wg
            rm = []
            for dw in range(GR):
                m = buf[p, w + dw, pl.ds(0, LANES)]
                for u in range(1, VPW):
                    m = jnp.maximum(m, buf[p, w + dw, pl.ds(u * LANES, LANES)])
                rm.append(m)
            gm = rm[0]
            for dw in range(1, GR):
                gm = jnp.maximum(gm, rm[dw])

            def hit(cc):
                for dw in range(GR):
                    def row_hit(c3, dw=dw):
                        for u in range(VPW):
                            v = buf[p, w + dw, pl.ds(u * LANES, LANES)]
                            c3 = lax.cond(_any_gt(v, c3[3]),
                                          functools.partial(insert, v),
                                          lambda c4: c4, c3)
                        return c3
                    cc = lax.cond(_any_gt(rm[dw], cc[3]), row_hit,
                                  lambda c3: c3, cc)
                return cc

            return lax.cond(_any_gt(gm, c[3]), hit, lambda c2: c2, c)

        carry = lax.fori_loop(0, W // GR, grp_body, carry)
    return carry


GROUPS = 1
GR_ROWS = ROWS // GROUPS
SHARDS = 32 // GR_ROWS
SH_PLANES = H // SHARDS
NPIECES = SH_PLANES // PLANES_PER_PIECE


def _sc_body(x_hbm, cand_hbm, buf0, buf1, out_v, sem0, sem1, *, g):
    wid = lax.axis_index("s") * 2 + lax.axis_index("c")
    row_local = wid // SHARDS
    shard = wid - row_local * SHARDS
    row = g * GR_ROWS + row_local
    base0 = shard * SH_PLANES

    def src(i):
        return x_hbm.at[row, pl.ds(base0 + i * PLANES_PER_PIECE,
                                   PLANES_PER_PIECE), :, :]

    pltpu.async_copy(src(0), buf0, sem0)
    pltpu.async_copy(src(1), buf1, sem1)

    neg = jnp.full((LANES,), NEG_INF, jnp.float32)
    carry0 = (neg, neg, neg, neg)

    def pair_body(gi, carry):
        i0 = 2 * gi
        pltpu.make_async_copy(src(i0), buf0, sem0).wait()
        carry = _process_piece(buf0, out_v, carry)

        @pl.when(i0 + 2 < NPIECES)
        def _():
            pltpu.async_copy(src(i0 + 2), buf0, sem0)

        pltpu.make_async_copy(src(i0 + 1), buf1, sem1).wait()
        carry = _process_piece(buf1, out_v, carry)

        @pl.when(i0 + 3 < NPIECES)
        def _():
            pltpu.async_copy(src(i0 + 3), buf1, sem1)

        return carry

    t0, t1, t2, _ = lax.fori_loop(0, NPIECES // 2, pair_body, carry0)
    out_v[pl.ds(0, LANES)] = jnp.full((LANES,), NEG_INF, jnp.float32)
    out_v[pl.ds(LANES, LANES)] = t0
    out_v[pl.ds(2 * LANES, LANES)] = t1
    out_v[pl.ds(3 * LANES, LANES)] = t2
    pltpu.sync_copy(out_v, cand_hbm.at[row_local, pl.ds(shard * 64, 64)])


def _sc_candidates(x4d, g):
    mesh = plsc.VectorSubcoreMesh(core_axis_name="c", subcore_axis_name="s")
    fn = pl.kernel(
        functools.partial(_sc_body, g=g),
        out_type=jax.ShapeDtypeStruct((GR_ROWS, SHARDS * 64), jnp.float32),
        mesh=mesh,
        compiler_params=pltpu.CompilerParams(needs_layout_passes=False,
                                             use_tc_tiling_on_sc=True),
        scratch_types=[
            pltpu.VMEM((PLANES_PER_PIECE, W, C), jnp.float32),
            pltpu.VMEM((PLANES_PER_PIECE, W, C), jnp.float32),
            pltpu.VMEM((64,), jnp.float32),
            pltpu.SemaphoreType.DMA,
            pltpu.SemaphoreType.DMA,
        ],
    )
    return fn(x4d)


H_BLK = 56
NSPLIT = 224 // H_BLK


NCVEC = SHARDS * 64 // 128


def _tc_mask_body(x_ref, c_ref, o_ref, tau_ref):
    j = pl.program_id(1)

    @pl.when(j == 0)
    def _():
        c = c_ref[0]
        u = lax.bitcast_convert_type(c, jnp.int32)
        skey = jnp.where(u >= 0, u, u ^ M31)
        t = jnp.int32(0)
        for b in range(31, -1, -1):
            cu = t | _i32(1 << b)
            scand = cu ^ MIN32
            cnt = jnp.sum((skey >= scand).astype(jnp.int32))
            t = jnp.where(cnt >= N_KEEP, cu, t)
        st = t ^ MIN32
        ub = jnp.where(st >= 0, st, st ^ M31)
        tauv = lax.bitcast_convert_type(
            jnp.broadcast_to(ub, (1, 128)), jnp.float32)
        tau_ref[...] = tauv

    tau = tau_ref[0, 0]
    x = x_ref[0]
    o_ref[0] = x * (x >= tau).astype(jnp.float32)


def _tc_mask_body_prev(x_ref, c_ref, prev_ref, o_ref, tau_ref):
    del prev_ref
    _tc_mask_body(x_ref, c_ref, o_ref, tau_ref)


def _tc_mask(x4d, cand3d, g, prev=None):
    in_specs = [
        pl.BlockSpec((1, H_BLK, 224, 96),
                     lambda r, j: (g * GR_ROWS + r, j, 0, 0)),
        pl.BlockSpec((1, NCVEC, 128), lambda r, j: (r, 0, 0)),
    ]
    args = [x4d, cand3d]
    body = _tc_mask_body
    io_alias = {}
    if prev is not None:
        in_specs.append(pl.BlockSpec(memory_space=pl.ANY))
        args.append(prev)
        body = _tc_mask_body_prev
        io_alias = {2: 0}
    return pl.pallas_call(
        body,
        grid=(GR_ROWS, NSPLIT),
        in_specs=in_specs,
        out_specs=pl.BlockSpec((1, H_BLK, 224, 96),
                               lambda r, j: (g * GR_ROWS + r, j, 0, 0)),
        out_shape=jax.ShapeDtypeStruct(x4d.shape, jnp.float32),
        scratch_shapes=[pltpu.VMEM((1, 128), jnp.float32)],
        input_output_aliases=io_alias,
    )(*args)


def kernel(inputs):
    cands = [_sc_candidates(inputs, g) for g in range(GROUPS)]
    full = None
    for g in range(GROUPS):
        c3 = cands[g].reshape(GR_ROWS, NCVEC, 128)
        full = _tc_mask(inputs, c3, g, prev=full)
    return full


# speedup vs baseline: 1.0592x; 1.0297x over previous
import functools

import jax
import jax.numpy as jnp
import numpy as np
from jax import lax
from jax.experimental import pallas as pl
from jax.experimental.pallas import tpu as pltpu
from jax.experimental.pallas import tpu_sc as plsc

N_KEEP = 48
ROWS = 16
H = 224
W = 224
C = 96
L = H * W * C
LANES = 16
VPW = C // LANES
PLANES_PER_PIECE = 2
NPIECES = (H // 2) // PLANES_PER_PIECE
NEG_INF = float("-inf")


def _i32(v):
    v &= 0xFFFFFFFF
    return np.int32(v - (1 << 32) if v >= (1 << 31) else v)


MIN32 = _i32(0x80000000)
M31 = _i32(0x7FFFFFFF)


def _bmerge(a, b):
    rb = jnp.flip(b)
    lo = jnp.minimum(a, rb)
    hi = jnp.maximum(a, rb)
    return jnp.sort(lo), jnp.sort(hi)


def _make_insert(th_v):
    def _insert(v, c):
        t0, t1, t2, _ = c
        s = jnp.sort(v)
        _, hi = _bmerge(s, t0)
        t0, hi = _bmerge(hi, t1)
        t1, t2 = _bmerge(hi, t2)
        theta = jnp.full((LANES,), t0[0], jnp.float32)
        return (t0, t1, t2, theta)
    return _insert


def _any_gt(v, theta):
    return plsc.all_reduce_population_count(v > theta)[0] > 0


def _process_piece(buf, th_v, carry):
    insert = _make_insert(th_v)

    GR = 4
    for p in range(PLANES_PER_PIECE):
        def grp_body(wg, c, p=p):
            w = GR * wg
            rm = []
            for dw in range(GR):
                m = buf[p, w + dw, pl.ds(0, LANES)]
                for u in range(1, VPW):
                    m = jnp.maximum(m, buf[p, w + dw, pl.ds(u * LANES, LANES)])
                rm.append(m)
            gm = rm[0]
            for dw in range(1, GR):
                gm = jnp.maximum(gm, rm[dw])

            def hit(cc):
                for dw in range(GR):
                    def row_hit(c3, dw=dw):
                        for u in range(VPW):
                            v = buf[p, w + dw, pl.ds(u * LANES, LANES)]
                            c3 = lax.cond(_any_gt(v, c3[3]),
                                          functools.partial(insert, v),
                                          lambda c4: c4, c3)
                        return c3
                    cc = lax.cond(_any_gt(rm[dw], cc[3]), row_hit,
                                  lambda c3: c3, cc)
                return cc

            return lax.cond(_any_gt(gm, c[3]), hit, lambda c2: c2, c)

        carry = lax.fori_loop(0, W // GR, grp_body, carry)
    return carry


GROUPS = 1
GR_ROWS = ROWS // GROUPS
SHARDS = 32 // GR_ROWS
SH_PLANES = H // SHARDS
NPIECES = SH_PLANES // PLANES_PER_PIECE


def _sc_body(x_hbm, cand_hbm, buf0, buf1, out_v, sem0, sem1, *, g):
    wid = lax.axis_index("s") * 2 + lax.axis_index("c")
    row_local = wid // SHARDS
    shard = wid - row_local * SHARDS
    row = g * GR_ROWS + row_local
    base0 = shard * SH_PLANES

    def src(i):
        return x_hbm.at[row, pl.ds(base0 + i * PLANES_PER_PIECE,
                                   PLANES_PER_PIECE), :, :]

    pltpu.async_copy(src(0), buf0, sem0)
    pltpu.async_copy(src(1), buf1, sem1)

    neg = jnp.full((LANES,), NEG_INF, jnp.float32)
    carry0 = (neg, neg, neg, neg)

    def pair_body(gi, carry):
        i0 = 2 * gi
        pltpu.make_async_copy(src(i0), buf0, sem0).wait()
        carry = _process_piece(buf0, out_v, carry)

        @pl.when(i0 + 2 < NPIECES)
        def _():
            pltpu.async_copy(src(i0 + 2), buf0, sem0)

        pltpu.make_async_copy(src(i0 + 1), buf1, sem1).wait()
        carry = _process_piece(buf1, out_v, carry)

        @pl.when(i0 + 3 < NPIECES)
        def _():
            pltpu.async_copy(src(i0 + 3), buf1, sem1)

        return carry

    t0, t1, t2, _ = lax.fori_loop(0, NPIECES // 2, pair_body, carry0)
    out_v[pl.ds(0, LANES)] = jnp.full((LANES,), NEG_INF, jnp.float32)
    out_v[pl.ds(LANES, LANES)] = t0
    out_v[pl.ds(2 * LANES, LANES)] = t1
    out_v[pl.ds(3 * LANES, LANES)] = t2
    pltpu.sync_copy(out_v, cand_hbm.at[row_local, pl.ds(shard * 64, 64)])


def _sc_candidates(x4d, g):
    mesh = plsc.VectorSubcoreMesh(core_axis_name="c", subcore_axis_name="s")
    fn = pl.kernel(
        functools.partial(_sc_body, g=g),
        out_type=jax.ShapeDtypeStruct((GR_ROWS, SHARDS * 64), jnp.float32),
        mesh=mesh,
        compiler_params=pltpu.CompilerParams(needs_layout_passes=False,
                                             use_tc_tiling_on_sc=True),
        scratch_types=[
            pltpu.VMEM((PLANES_PER_PIECE, W, C), jnp.float32),
            pltpu.VMEM((PLANES_PER_PIECE, W, C), jnp.float32),
            pltpu.VMEM((64,), jnp.float32),
            pltpu.SemaphoreType.DMA,
            pltpu.SemaphoreType.DMA,
        ],
    )
    return fn(x4d)


H_BLK = 112
NSPLIT = 224 // H_BLK


NCVEC = SHARDS * 64 // 128


def _tc_mask_body(x_ref, c_ref, o_ref, tau_ref):
    j = pl.program_id(1)

    @pl.when(j == 0)
    def _():
        c = c_ref[0]
        u = lax.bitcast_convert_type(c, jnp.int32)
        skey = jnp.where(u >= 0, u, u ^ M31)
        t = jnp.int32(0)
        for b in range(31, -1, -1):
            cu = t | _i32(1 << b)
            scand = cu ^ MIN32
            cnt = jnp.sum((skey >= scand).astype(jnp.int32))
            t = jnp.where(cnt >= N_KEEP, cu, t)
        st = t ^ MIN32
        ub = jnp.where(st >= 0, st, st ^ M31)
        tauv = lax.bitcast_convert_type(
            jnp.broadcast_to(ub, (1, 128)), jnp.float32)
        tau_ref[...] = tauv

    tau = tau_ref[0, 0]
    x = x_ref[0]
    o_ref[0] = x * (x >= tau).astype(jnp.float32)


def _tc_mask_body_prev(x_ref, c_ref, prev_ref, o_ref, tau_ref):
    del prev_ref
    _tc_mask_body(x_ref, c_ref, o_ref, tau_ref)


def _tc_mask(x4d, cand3d, g, prev=None):
    in_specs = [
        pl.BlockSpec((1, H_BLK, 224, 96),
                     lambda r, j: (g * GR_ROWS + r, j, 0, 0)),
        pl.BlockSpec((1, NCVEC, 128), lambda r, j: (r, 0, 0)),
    ]
    args = [x4d, cand3d]
    body = _tc_mask_body
    io_alias = {}
    if prev is not None:
        in_specs.append(pl.BlockSpec(memory_space=pl.ANY))
        args.append(prev)
        body = _tc_mask_body_prev
        io_alias = {2: 0}
    return pl.pallas_call(
        body,
        grid=(GR_ROWS, NSPLIT),
        in_specs=in_specs,
        out_specs=pl.BlockSpec((1, H_BLK, 224, 96),
                               lambda r, j: (g * GR_ROWS + r, j, 0, 0)),
        out_shape=jax.ShapeDtypeStruct(x4d.shape, jnp.float32),
        scratch_shapes=[pltpu.VMEM((1, 128), jnp.float32)],
        input_output_aliases=io_alias,
        compiler_params=pltpu.CompilerParams(
            vmem_limit_bytes=100 * 1024 * 1024),
    )(*args)


def kernel(inputs):
    cands = [_sc_candidates(inputs, g) for g in range(GROUPS)]
    full = None
    for g in range(GROUPS):
        c3 = cands[g].reshape(GR_ROWS, NCVEC, 128)
        full = _tc_mask(inputs, c3, g, prev=full)
    return full
